# Initial kernel scaffold; baseline (speedup 1.0000x reference)
import jax
import jax.numpy as jnp
from jax.experimental import pallas as pl


def kernel(rel_emb, length):
    raise NotImplementedError("write your pallas kernel here")


# speedup vs baseline: 1.4090x; 1.4090x over previous
import functools

import jax
import jax.numpy as jnp
from jax import lax
from jax.experimental import pallas as pl
from jax.experimental.pallas import tpu as pltpu
from jax.experimental.pallas import tpu_sc as plsc

D_MODEL = 768
MAX_LEN = 2048
SEQ = 512
WIN = 1024
W0 = MAX_LEN - SEQ

NUM_CORES = 2
NUM_SUBCORES = 16
NUM_WORKERS = NUM_CORES * NUM_SUBCORES
ROWS_PER_WORKER = SEQ // NUM_WORKERS
STAGE_ROWS = WIN // NUM_SUBCORES


@functools.partial(
    pl.kernel,
    mesh=plsc.VectorSubcoreMesh(core_axis_name="c", subcore_axis_name="s"),
    out_type=jax.ShapeDtypeStruct((SEQ * SEQ * D_MODEL,), jnp.float32),
    scratch_types=[
        pltpu.VMEM_SHARED((WIN * D_MODEL,), jnp.float32),
        pltpu.SemaphoreType.DMA,
    ],
)
def _rpe_sc(rel_hbm, out_hbm, window, sem):
    c = lax.axis_index("c")
    s = lax.axis_index("s")

    pltpu.sync_copy(
        rel_hbm.at[pl.ds((W0 + s * STAGE_ROWS) * D_MODEL, STAGE_ROWS * D_MODEL)],
        window.at[pl.ds(s * STAGE_ROWS * D_MODEL, STAGE_ROWS * D_MODEL)],
    )
    plsc.subcore_barrier()

    wid = s * NUM_CORES + c
    base = wid * ROWS_PER_WORKER
    copies = []
    for r in range(ROWS_PER_WORKER):
        i = base + r
        copies.append(
            pltpu.async_copy(
                window.at[pl.ds((SEQ - 1 - i) * D_MODEL, SEQ * D_MODEL)],
                out_hbm.at[pl.ds(i * SEQ * D_MODEL, SEQ * D_MODEL)],
                sem,
            )
        )
    for cp in copies:
        cp.wait()


def kernel(rel_emb, length):
    del length
    flat = _rpe_sc(rel_emb.reshape(-1))
    return flat.reshape(SEQ, SEQ, D_MODEL)
